# baseline (speedup 1.0000x reference)
import functools

import jax
import jax.numpy as jnp
from jax import lax
from jax.experimental import pallas as pl
from jax.experimental.pallas import tpu as pltpu
from jax.experimental.pallas import tpu_sc as plsc

_N = 375
_TOTAL = _N * _N
_NW = 32
_CHUNK = 4400
_PADDED = _NW * _CHUNK
_VECS = _CHUNK // 16


def _make_sc_gather():
    mesh = plsc.VectorSubcoreMesh(core_axis_name="c", subcore_axis_name="s")

    @functools.partial(
        pl.kernel,
        mesh=mesh,
        out_type=jax.ShapeDtypeStruct((_PADDED,), jnp.float32),
        scratch_types=[
            pltpu.VMEM((16,), jnp.float32),
            pltpu.VMEM((_CHUNK,), jnp.int32),
            pltpu.VMEM((_CHUNK,), jnp.float32),
        ],
    )
    def gather_kernel(idx_hbm, tab_hbm, out_hbm, tab_v, idx_v, val_v):
        wid = lax.axis_index("s") * 2 + lax.axis_index("c")
        base = wid * _CHUNK
        pltpu.sync_copy(tab_hbm, tab_v)
        pltpu.sync_copy(idx_hbm.at[pl.ds(base, _CHUNK)], idx_v)
        tab_vec = tab_v[...]

        dnums = lax.GatherDimensionNumbers(
            offset_dims=(), collapsed_slice_dims=(0,), start_index_map=(0,)
        )

        def body(i, carry):
            idx16 = idx_v[pl.ds(i * 16, 16)]
            val_v[pl.ds(i * 16, 16)] = lax.gather(
                tab_vec,
                idx16[:, None],
                dnums,
                slice_sizes=(1,),
                mode=lax.GatherScatterMode.PROMISE_IN_BOUNDS,
            )
            return carry

        lax.fori_loop(0, _VECS, body, 0)
        pltpu.sync_copy(val_v, out_hbm.at[pl.ds(base, _CHUNK)])

    return gather_kernel


_sc_gather = _make_sc_gather()


def kernel(dist_par_0, par_table0):
    flat = jnp.pad(dist_par_0.reshape(-1), (0, _PADDED - _TOTAL))
    tab = jnp.pad(par_table0.reshape(-1), (0, 2))
    out = _sc_gather(flat, tab)
    return out[:_TOTAL].reshape(_N, _N)


# speedup vs baseline: 1.0239x; 1.0239x over previous
import functools

import jax
import jax.numpy as jnp
from jax import lax
from jax.experimental import pallas as pl
from jax.experimental.pallas import tpu as pltpu
from jax.experimental.pallas import tpu_sc as plsc

_N = 375
_TOTAL = _N * _N
_NW = 32
_CHUNK = 4400
_LAST_BASE = 31 * _CHUNK
_LAST = _TOTAL - _LAST_BASE
_VECS = _CHUNK // 16
_LAST_VECS = -(-_LAST // 16)


def _make_sc_gather():
    mesh = plsc.VectorSubcoreMesh(core_axis_name="c", subcore_axis_name="s")

    @functools.partial(
        pl.kernel,
        mesh=mesh,
        out_type=jax.ShapeDtypeStruct((_TOTAL,), jnp.float32),
        scratch_types=[
            pltpu.VMEM((16,), jnp.float32),
            pltpu.VMEM((_CHUNK,), jnp.int32),
            pltpu.VMEM((_CHUNK,), jnp.float32),
            pltpu.SemaphoreType.DMA,
            pltpu.SemaphoreType.DMA,
        ],
    )
    def gather_kernel(idx_hbm, tab_hbm, out_hbm, tab_v, idx_v, val_v, sem0, sem1):
        wid = lax.axis_index("s") * 2 + lax.axis_index("c")
        base = wid * _CHUNK
        tab_cp = pltpu.async_copy(tab_hbm, tab_v, sem0)

        dnums = lax.GatherDimensionNumbers(
            offset_dims=(), collapsed_slice_dims=(0,), start_index_map=(0,)
        )

        def run(n_elems, n_vecs):
            idx_cp = pltpu.async_copy(
                idx_hbm.at[pl.ds(base, n_elems)], idx_v.at[pl.ds(0, n_elems)], sem1
            )
            tab_cp.wait()
            tab_vec = tab_v[...]
            idx_cp.wait()

            def body(i, carry):
                idx16 = idx_v[pl.ds(i * 16, 16)]
                val_v[pl.ds(i * 16, 16)] = lax.gather(
                    tab_vec,
                    idx16[:, None],
                    dnums,
                    slice_sizes=(1,),
                    mode=lax.GatherScatterMode.PROMISE_IN_BOUNDS,
                )
                return carry

            lax.fori_loop(0, n_vecs, body, 0)
            pltpu.sync_copy(
                val_v.at[pl.ds(0, n_elems)], out_hbm.at[pl.ds(base, n_elems)]
            )

        @pl.when(wid < _NW - 1)
        def _():
            run(_CHUNK, _VECS)

        @pl.when(wid == _NW - 1)
        def _():
            idx_v[pl.ds((_LAST_VECS - 1) * 16, 16)] = jnp.zeros((16,), jnp.int32)
            run(_LAST, _LAST_VECS)

    return gather_kernel


_sc_gather = _make_sc_gather()


def kernel(dist_par_0, par_table0):
    flat = dist_par_0.reshape(-1)
    tab = jnp.pad(par_table0.reshape(-1), (0, 2))
    out = _sc_gather(flat, tab)
    return out.reshape(_N, _N)
